# baseline (speedup 1.0000x reference)
import functools

import jax
import jax.numpy as jnp
from jax import lax
from jax.experimental import pallas as pl
from jax.experimental.pallas import tpu as pltpu
from jax.experimental.pallas import tpu_sc as plsc

NUM_CORES = 2
NUM_SUBCORES = 16
NUM_WORKERS = NUM_CORES * NUM_SUBCORES
DEPTH = 5
PAD = 128


@functools.partial(jax.jit, static_argnums=(2, 3, 4))
def _sc_gather_t(table, idx_t, B, T, feat):
    bpw = B // NUM_WORKERS
    mesh = plsc.VectorSubcoreMesh(core_axis_name="c", subcore_axis_name="s")

    @functools.partial(
        pl.kernel,
        mesh=mesh,
        compiler_params=pltpu.CompilerParams(use_tc_tiling_on_sc=False),
        out_type=jax.ShapeDtypeStruct((B, T, PAD), jnp.int32),
        scratch_types=[
            pltpu.VMEM((T, bpw), jnp.int32),
            pltpu.VMEM((DEPTH, bpw, feat), jnp.int32),
        ]
        + [pltpu.SemaphoreType.DMA] * (2 * DEPTH),
    )
    def run(table_hbm, idx_hbm, out_hbm, idx_v, rows_v, *sems):
        gsems = sems[:DEPTH]
        wsems = sems[DEPTH:]
        wid = lax.axis_index("s") * NUM_CORES + lax.axis_index("c")
        b0 = wid * bpw
        pltpu.sync_copy(idx_hbm.at[:, pl.ds(b0, bpw)], idx_v)

        for d in range(DEPTH - 1):
            pltpu.async_copy(table_hbm.at[idx_v.at[d]], rows_v.at[d], gsems[d])

        def outer(o, _):
            for d in range(DEPTH):
                t = o * DEPTH + d
                prev = (d - 1) % DEPTH

                @pl.when(t > 0)
                def _():
                    pltpu.make_async_copy(
                        rows_v.at[prev],
                        out_hbm.at[pl.ds(b0, bpw), t - 1, pl.ds(0, feat)],
                        wsems[prev],
                    ).wait()

                @pl.when(t + DEPTH - 1 < T)
                def _():
                    pltpu.async_copy(
                        table_hbm.at[idx_v.at[t + DEPTH - 1]],
                        rows_v.at[prev],
                        gsems[prev],
                    )

                pltpu.make_async_copy(
                    table_hbm.at[idx_v.at[t]], rows_v.at[d], gsems[d]
                ).wait()
                pltpu.async_copy(
                    rows_v.at[d],
                    out_hbm.at[pl.ds(b0, bpw), t, pl.ds(0, feat)],
                    wsems[d],
                )
            return 0

        lax.fori_loop(0, T // DEPTH, outer, 0)
        last = (T - 1) % DEPTH
        pltpu.make_async_copy(
            rows_v.at[last],
            out_hbm.at[pl.ds(b0, bpw), T - 1, pl.ds(0, feat)],
            wsems[last],
        ).wait()

    return run(table, idx_t)


def kernel(token_ids, W):
    B, T = token_ids.shape
    V, feat = W.shape
    assert B % (NUM_WORKERS * 128) == 0 and T % DEPTH == 0
    table_i32 = W.astype(jnp.int32)
    idx_t = token_ids.astype(jnp.int32).T
    padded = _sc_gather_t(table_i32, idx_t, B, T, feat)
    return padded[:, :, :feat].astype(jnp.int64)
